# baseline (speedup 1.0000x reference)
import functools
import math

import jax
import jax.numpy as jnp
from jax import lax
from jax.experimental import pallas as pl
from jax.experimental.pallas import tpu as pltpu
from jax.experimental.pallas import tpu_sc as plsc

D_MODEL = 64
SCALE = math.sqrt(D_MODEL)
LANES = 16
W = 128
NB = 4
NW = 32


def kernel(x, table):
    b, s = x.shape
    n_items = b // W * s
    per_tile = n_items // NW
    rounds = per_tile // NB
    ib_per_j = b // W
    mesh = plsc.VectorSubcoreMesh(core_axis_name="core", subcore_axis_name="subcore")

    @functools.partial(
        pl.kernel,
        out_type=jax.ShapeDtypeStruct((s, D_MODEL, b), table.dtype),
        mesh=mesh,
        compiler_params=pltpu.CompilerParams(
            use_tc_tiling_on_sc=False, needs_layout_passes=False),
        scratch_types=(
            [pltpu.VMEM((W,), jnp.int32) for _ in range(NB)]
            + [pltpu.VMEM((W, D_MODEL), jnp.float32) for _ in range(NB)]
            + [pltpu.VMEM((D_MODEL, W), jnp.float32) for _ in range(NB)]
            + [pltpu.SemaphoreType.DMA for _ in range(3 * NB)]
        ),
    )
    def run(table_hbm, xt_hbm, out_hbm, *scr):
        idxb = scr[:NB]
        gbuf = scr[NB:2 * NB]
        tbuf = scr[2 * NB:3 * NB]
        isem = scr[3 * NB:4 * NB]
        gsem = scr[4 * NB:5 * NB]
        wsem = scr[5 * NB:6 * NB]
        wid = lax.axis_index("core") * 16 + lax.axis_index("subcore")
        m0 = wid * per_tile

        iota16 = lax.iota(jnp.int32, LANES)
        row_ids = [k * LANES + iota16 for k in range(W // LANES)]

        def decode(m):
            j = m // ib_per_j
            i0 = (m % ib_per_j) * W
            return j, i0

        def idx_start(bf, m):
            j, i0 = decode(m)
            pltpu.make_async_copy(
                xt_hbm.at[j, pl.ds(i0, W)], idxb[bf], isem[bf]).start()

        def idx_wait(bf, m):
            j, i0 = decode(m)
            pltpu.make_async_copy(
                xt_hbm.at[j, pl.ds(i0, W)], idxb[bf], isem[bf]).wait()

        def gather_start(bf):
            pltpu.make_async_copy(
                table_hbm.at[idxb[bf]], gbuf[bf], gsem[bf]).start()

        def gather_wait(bf):
            pltpu.make_async_copy(
                table_hbm.at[idxb[bf]], gbuf[bf], gsem[bf]).wait()

        def write_start(bf, m):
            j, i0 = decode(m)
            pltpu.make_async_copy(
                tbuf[bf], out_hbm.at[j, :, pl.ds(i0, W)], wsem[bf]).start()

        def write_wait(bf, m):
            j, i0 = decode(m)
            pltpu.make_async_copy(
                tbuf[bf], out_hbm.at[j, :, pl.ds(i0, W)], wsem[bf]).wait()

        def transpose_scale(bf):
            src, dst = gbuf[bf], tbuf[bf]

            @pl.loop(0, D_MODEL)
            def _(d):
                col = jnp.full((LANES,), 0, jnp.int32) + d
                for k in range(W // LANES):
                    v = plsc.load_gather(src, [row_ids[k], col])
                    dst.at[d, pl.ds(k * LANES, LANES)][...] = v * SCALE

        for bf in range(NB):
            idx_start(bf, m0 + bf)
        for bf in range(NB):
            idx_wait(bf, m0 + bf)
            gather_start(bf)

        for bf in range(NB):
            m = m0 + bf
            gather_wait(bf)
            idx_start(bf, m + NB)
            transpose_scale(bf)
            write_start(bf, m)
            idx_wait(bf, m + NB)
            gather_start(bf)

        @pl.loop(1, rounds)
        def _(r):
            for bf in range(NB):
                m = m0 + r * NB + bf
                gather_wait(bf)

                @pl.when(r + 1 < rounds)
                def _():
                    idx_start(bf, m + NB)

                write_wait(bf, m - NB)
                transpose_scale(bf)
                write_start(bf, m)

                @pl.when(r + 1 < rounds)
                def _():
                    idx_wait(bf, m + NB)
                    gather_start(bf)

        for bf in range(NB):
            write_wait(bf, m0 + per_tile - NB + bf)

    out_t = run(table, x.astype(jnp.int32).T)
    return jnp.transpose(out_t, (2, 0, 1))


# speedup vs baseline: 1.0934x; 1.0934x over previous
import functools
import math

import jax
import jax.numpy as jnp
from jax import lax
from jax.experimental import pallas as pl
from jax.experimental.pallas import tpu as pltpu
from jax.experimental.pallas import tpu_sc as plsc

D_MODEL = 64
SCALE = math.sqrt(D_MODEL)
LANES = 16
W = 128
NB = 4
NW = 32


def kernel(x, table):
    b, s = x.shape
    v = table.shape[0]
    ib_per_j = b // W
    n_items = ib_per_j * s
    per_tile = n_items // NW
    rounds = per_tile // NB
    mesh = plsc.VectorSubcoreMesh(core_axis_name="core", subcore_axis_name="subcore")

    @functools.partial(
        pl.kernel,
        out_type=jax.ShapeDtypeStruct((s * D_MODEL, b), table.dtype),
        mesh=mesh,
        compiler_params=pltpu.CompilerParams(needs_layout_passes=False),
        scratch_types=(
            [pltpu.VMEM((8, W), jnp.int32) for _ in range(NB)]
            + [pltpu.VMEM((W,), jnp.int32) for _ in range(NB)]
            + [pltpu.VMEM((8, LANES), jnp.int32) for _ in range(NB)]
            + [pltpu.VMEM((W, 2 * D_MODEL), jnp.float32) for _ in range(NB)]
            + [pltpu.VMEM((D_MODEL, W), jnp.float32) for _ in range(NB)]
            + [pltpu.SemaphoreType.DMA for _ in range(3 * NB)]
        ),
    )
    def run(table_hbm, xt_hbm, out_hbm, *scr):
        idxr = scr[:NB]
        idx2 = scr[NB:2 * NB]
        cbas = scr[2 * NB:3 * NB]
        gbuf = scr[3 * NB:4 * NB]
        tbuf = scr[4 * NB:5 * NB]
        isem = scr[5 * NB:6 * NB]
        gsem = scr[6 * NB:7 * NB]
        wsem = scr[7 * NB:8 * NB]
        wid = lax.axis_index("core") * 16 + lax.axis_index("subcore")
        m0 = wid * per_tile

        iota16 = lax.iota(jnp.int32, LANES)
        row_ids = [k * LANES + iota16 for k in range(W // LANES)]

        def decode(m):
            j = m // ib_per_j
            i0 = (m % ib_per_j) * W
            return j, i0

        def idx_start(bf, m):
            j, i0 = decode(m)
            jb = (j // 8) * 8
            pltpu.make_async_copy(
                xt_hbm.at[pl.ds(jb, 8), pl.ds(i0, W)], idxr[bf], isem[bf]).start()

        def idx_finish(bf, m):
            j, i0 = decode(m)
            jb = (j // 8) * 8
            pltpu.make_async_copy(
                xt_hbm.at[pl.ds(jb, 8), pl.ds(i0, W)], idxr[bf], isem[bf]).wait()
            jr = j - jb
            for k in range(W // LANES):
                raw = idxr[bf].at[jr, pl.ds(k * LANES, LANES)][...]
                idx2[bf].at[pl.ds(k * LANES, LANES)][...] = raw >> 1
                cbas[bf].at[k][...] = (raw & 1) * D_MODEL

        def gather_start(bf):
            pltpu.make_async_copy(
                table_hbm.at[idx2[bf]], gbuf[bf], gsem[bf]).start()

        def gather_wait(bf):
            pltpu.make_async_copy(
                table_hbm.at[idx2[bf]], gbuf[bf], gsem[bf]).wait()

        def write_start(bf, m):
            j, i0 = decode(m)
            pltpu.make_async_copy(
                tbuf[bf], out_hbm.at[pl.ds(j * D_MODEL, D_MODEL), pl.ds(i0, W)],
                wsem[bf]).start()

        def write_wait(bf, m):
            j, i0 = decode(m)
            pltpu.make_async_copy(
                tbuf[bf], out_hbm.at[pl.ds(j * D_MODEL, D_MODEL), pl.ds(i0, W)],
                wsem[bf]).wait()

        def transpose_scale(bf):
            src, dst = gbuf[bf], tbuf[bf]
            col0 = [cbas[bf].at[k][...] for k in range(W // LANES)]

            @pl.loop(0, D_MODEL, step=8)
            def _(d0):
                for dd in range(8):
                    for k in range(W // LANES):
                        vec = plsc.load_gather(src, [row_ids[k], col0[k] + (d0 + dd)])
                        dst.at[d0 + dd, pl.ds(k * LANES, LANES)][...] = vec * SCALE

        for bf in range(NB):
            idx_start(bf, m0 + bf)
        for bf in range(NB):
            idx_finish(bf, m0 + bf)
            gather_start(bf)

        @pl.loop(0, rounds)
        def _(r):
            for bf in range(NB):
                m = m0 + r * NB + bf
                gather_wait(bf)

                @pl.when(r + 1 < rounds)
                def _():
                    idx_start(bf, m + NB)

                @pl.when(r > 0)
                def _():
                    write_wait(bf, m - NB)

                transpose_scale(bf)
                write_start(bf, m)

                @pl.when(r + 1 < rounds)
                def _():
                    idx_finish(bf, m + NB)
                    gather_start(bf)

        for bf in range(NB):
            write_wait(bf, m0 + per_tile - NB + bf)

    out2 = run(table.reshape(v // 2, 2 * D_MODEL), x.astype(jnp.int32).T)
    return jnp.transpose(out2.reshape(s, D_MODEL, b), (2, 0, 1))


# speedup vs baseline: 1.5687x; 1.4346x over previous
import functools
import math

import jax
import jax.numpy as jnp
from jax import lax
from jax.experimental import pallas as pl
from jax.experimental.pallas import tpu as pltpu
from jax.experimental.pallas import tpu_sc as plsc

D_MODEL = 64
SCALE = math.sqrt(D_MODEL)
LANES = 16
W = 128
NB = 4
NW = 32


def kernel(x, table):
    b, s = x.shape
    v = table.shape[0]
    ib_per_j = b // W
    n_items = ib_per_j * s
    per_tile = n_items // NW
    rounds = per_tile // NB
    mesh = plsc.VectorSubcoreMesh(core_axis_name="core", subcore_axis_name="subcore")

    @functools.partial(
        pl.kernel,
        out_type=jax.ShapeDtypeStruct((s * D_MODEL, b), table.dtype),
        mesh=mesh,
        compiler_params=pltpu.CompilerParams(needs_layout_passes=False),
        scratch_types=(
            [pltpu.VMEM((8, W), jnp.int32) for _ in range(NB)]
            + [pltpu.VMEM((W,), jnp.int32) for _ in range(NB)]
            + [pltpu.VMEM((8, LANES), jnp.int32) for _ in range(NB)]
            + [pltpu.VMEM((W, 2 * D_MODEL), jnp.float32) for _ in range(NB)]
            + [pltpu.VMEM((D_MODEL, W), jnp.float32) for _ in range(NB)]
            + [pltpu.SemaphoreType.DMA for _ in range(3 * NB)]
        ),
    )
    def run(table_hbm, xt_hbm, out_hbm, *scr):
        idxr = scr[:NB]
        idx2 = scr[NB:2 * NB]
        cbas = scr[2 * NB:3 * NB]
        gbuf = scr[3 * NB:4 * NB]
        tbuf = scr[4 * NB:5 * NB]
        isem = scr[5 * NB:6 * NB]
        gsem = scr[6 * NB:7 * NB]
        wsem = scr[7 * NB:8 * NB]
        wid = lax.axis_index("core") * 16 + lax.axis_index("subcore")
        m0 = wid * per_tile

        iota16 = lax.iota(jnp.int32, LANES)
        row_ids = [k * LANES + iota16 for k in range(W // LANES)]

        def decode(m):
            j = m // ib_per_j
            i0 = (m % ib_per_j) * W
            return j, i0

        def idx_start(bf, m):
            j, i0 = decode(m)
            jb = (j // 8) * 8
            pltpu.make_async_copy(
                xt_hbm.at[pl.ds(jb, 8), pl.ds(i0, W)], idxr[bf], isem[bf]).start()

        def idx_finish(bf, m):
            j, i0 = decode(m)
            jb = (j // 8) * 8
            pltpu.make_async_copy(
                xt_hbm.at[pl.ds(jb, 8), pl.ds(i0, W)], idxr[bf], isem[bf]).wait()
            jr = j - jb
            for k in range(W // LANES):
                raw = idxr[bf].at[jr, pl.ds(k * LANES, LANES)][...]
                idx2[bf].at[pl.ds(k * LANES, LANES)][...] = raw >> 1
                cbas[bf].at[k][...] = (raw & 1) * D_MODEL

        def gather_start(bf):
            pltpu.make_async_copy(
                table_hbm.at[idx2[bf]], gbuf[bf], gsem[bf]).start()

        def gather_wait(bf):
            pltpu.make_async_copy(
                table_hbm.at[idx2[bf]], gbuf[bf], gsem[bf]).wait()

        def write_start(bf, m):
            j, i0 = decode(m)
            pltpu.make_async_copy(
                tbuf[bf], out_hbm.at[pl.ds(j * D_MODEL, D_MODEL), pl.ds(i0, W)],
                wsem[bf]).start()

        def write_wait(bf, m):
            j, i0 = decode(m)
            pltpu.make_async_copy(
                tbuf[bf], out_hbm.at[pl.ds(j * D_MODEL, D_MODEL), pl.ds(i0, W)],
                wsem[bf]).wait()

        def transpose_scale(bf):
            src, dst = gbuf[bf], tbuf[bf]
            col0 = [cbas[bf].at[k][...] for k in range(W // LANES)]

            @plsc.parallel_loop(0, D_MODEL, step=8)
            def _(d0):
                for dd in range(8):
                    for k in range(W // LANES):
                        vec = plsc.load_gather(src, [row_ids[k], col0[k] + (d0 + dd)])
                        dst.at[d0 + dd, pl.ds(k * LANES, LANES)][...] = vec * SCALE

        for bf in range(NB):
            idx_start(bf, m0 + bf)
        for bf in range(NB):
            idx_finish(bf, m0 + bf)
            gather_start(bf)

        @pl.loop(0, rounds)
        def _(r):
            for bf in range(NB):
                m = m0 + r * NB + bf
                gather_wait(bf)

                @pl.when(r + 1 < rounds)
                def _():
                    idx_start(bf, m + NB)

                @pl.when(r > 0)
                def _():
                    write_wait(bf, m - NB)

                transpose_scale(bf)
                write_start(bf, m)

                @pl.when(r + 1 < rounds)
                def _():
                    idx_finish(bf, m + NB)
                    gather_start(bf)

        for bf in range(NB):
            write_wait(bf, m0 + per_tile - NB + bf)

    out2 = run(table.reshape(v // 2, 2 * D_MODEL), x.astype(jnp.int32).T)
    return jnp.transpose(out2.reshape(s, D_MODEL, b), (2, 0, 1))


# speedup vs baseline: 2.4544x; 1.5647x over previous
import functools
import math

import jax
import jax.numpy as jnp
from jax import lax
from jax.experimental import pallas as pl
from jax.experimental.pallas import tpu as pltpu
from jax.experimental.pallas import tpu_sc as plsc

D_MODEL = 64
SCALE = math.sqrt(D_MODEL)
LANES = 16
W = 128
NB = 4
NW = 32


def kernel(x, table):
    b, s = x.shape
    v = table.shape[0]
    ib_per_j = b // W
    n_items = ib_per_j * s
    per_tile = n_items // NW
    rounds = per_tile // NB
    mesh = plsc.VectorSubcoreMesh(core_axis_name="core", subcore_axis_name="subcore")

    @functools.partial(
        pl.kernel,
        out_type=jax.ShapeDtypeStruct((s * D_MODEL, b), table.dtype),
        mesh=mesh,
        compiler_params=pltpu.CompilerParams(needs_layout_passes=False),
        scratch_types=(
            [pltpu.VMEM((8, W), jnp.int32) for _ in range(NB)]
            + [pltpu.VMEM((W,), jnp.int32) for _ in range(NB)]
            + [pltpu.VMEM((8, LANES), jnp.int32) for _ in range(NB)]
            + [pltpu.VMEM((W, 2 * D_MODEL), jnp.float32) for _ in range(NB)]
            + [pltpu.VMEM((D_MODEL, W), jnp.float32) for _ in range(NB)]
            + [pltpu.SemaphoreType.DMA for _ in range(3 * NB)]
        ),
    )
    def run(table_hbm, xt_hbm, out_hbm, *scr):
        idxr = scr[:NB]
        idx2 = scr[NB:2 * NB]
        cbas = scr[2 * NB:3 * NB]
        gbuf = scr[3 * NB:4 * NB]
        tbuf = scr[4 * NB:5 * NB]
        isem = scr[5 * NB:6 * NB]
        gsem = scr[6 * NB:7 * NB]
        wsem = scr[7 * NB:8 * NB]
        wid = lax.axis_index("core") * 16 + lax.axis_index("subcore")
        m0 = wid * per_tile

        iota16 = lax.iota(jnp.int32, LANES)
        row_ids = [k * LANES + iota16 for k in range(W // LANES)]
        rot = [(iota16 + s2) & (LANES - 1) for s2 in range(LANES)]

        def decode(m):
            j = m // ib_per_j
            i0 = (m % ib_per_j) * W
            return j, i0

        def idx_start(bf, m):
            j, i0 = decode(m)
            jb = (j // 8) * 8
            pltpu.make_async_copy(
                xt_hbm.at[pl.ds(jb, 8), pl.ds(i0, W)], idxr[bf], isem[bf]).start()

        def idx_finish(bf, m):
            j, i0 = decode(m)
            jb = (j // 8) * 8
            pltpu.make_async_copy(
                xt_hbm.at[pl.ds(jb, 8), pl.ds(i0, W)], idxr[bf], isem[bf]).wait()
            jr = j - jb
            for k in range(W // LANES):
                raw = idxr[bf].at[jr, pl.ds(k * LANES, LANES)][...]
                idx2[bf].at[pl.ds(k * LANES, LANES)][...] = raw >> 1
                cbas[bf].at[k][...] = (raw & 1) * D_MODEL

        def gather_start(bf):
            pltpu.make_async_copy(
                table_hbm.at[idx2[bf]], gbuf[bf], gsem[bf]).start()

        def gather_wait(bf):
            pltpu.make_async_copy(
                table_hbm.at[idx2[bf]], gbuf[bf], gsem[bf]).wait()

        def write_start(bf, m):
            j, i0 = decode(m)
            pltpu.make_async_copy(
                tbuf[bf], out_hbm.at[pl.ds(j * D_MODEL, D_MODEL), pl.ds(i0, W)],
                wsem[bf]).start()

        def write_wait(bf, m):
            j, i0 = decode(m)
            pltpu.make_async_copy(
                tbuf[bf], out_hbm.at[pl.ds(j * D_MODEL, D_MODEL), pl.ds(i0, W)],
                wsem[bf]).wait()

        def transpose_scale(bf):
            src, dst = gbuf[bf], tbuf[bf]

            @pl.loop(0, W // LANES)
            def _(k):
                rowv = k * LANES + iota16
                ck = cbas[bf].at[k][...]

                @plsc.parallel_loop(0, D_MODEL, step=LANES)
                def _(d0):
                    base = ck + d0
                    for s2 in range(LANES):
                        vec = plsc.load_gather(src, [rowv, base + rot[s2]])
                        plsc.store_scatter(dst, [d0 + rot[s2], rowv],
                                           vec * SCALE)

        for bf in range(NB):
            idx_start(bf, m0 + bf)
        for bf in range(NB):
            idx_finish(bf, m0 + bf)
            gather_start(bf)

        @pl.loop(0, rounds)
        def _(r):
            for bf in range(NB):
                m = m0 + r * NB + bf
                gather_wait(bf)

                @pl.when(r + 1 < rounds)
                def _():
                    idx_start(bf, m + NB)

                @pl.when(r > 0)
                def _():
                    write_wait(bf, m - NB)

                transpose_scale(bf)
                write_start(bf, m)

                @pl.when(r + 1 < rounds)
                def _():
                    idx_finish(bf, m + NB)
                    gather_start(bf)

        for bf in range(NB):
            write_wait(bf, m0 + per_tile - NB + bf)

    out2 = run(table.reshape(v // 2, 2 * D_MODEL), x.astype(jnp.int32).T)
    return jnp.transpose(out2.reshape(s, D_MODEL, b), (2, 0, 1))
